# Initial kernel scaffold; baseline (speedup 1.0000x reference)
import jax
import jax.numpy as jnp
from jax.experimental import pallas as pl


def kernel(tokens, embedding_weights):
    raise NotImplementedError("write your pallas kernel here")


# speedup vs baseline: 4.8089x; 4.8089x over previous
import functools

import jax
import jax.numpy as jnp
from jax import lax
from jax.experimental import pallas as pl
from jax.experimental import layout as jex_layout
from jax.experimental.pallas import tpu as pltpu
from jax.experimental.pallas import tpu_sc as plsc

_NC = 2
_NS = 16
_NW = _NC * _NS

_B = 16384 * 200
_D = 32
_BPW = _B // _NW
_C = 1024
_NCHUNK = _BPW // _C


def _gather_body(tokens_hbm, table_hbm, out_hbm, idx_v, rows_v, sem):
    wid = lax.axis_index("s") * _NC + lax.axis_index("c")
    base = wid * _BPW

    def chunk(i, carry):
        off = pl.multiple_of(base + i * _C, _C)
        pltpu.sync_copy(tokens_hbm.at[pl.ds(off, _C)], idx_v)
        pltpu.async_copy(table_hbm.at[idx_v], rows_v, sem).wait()
        pltpu.sync_copy(rows_v, out_hbm.at[pl.ds(off, _C)])
        return carry

    lax.fori_loop(0, _NCHUNK, chunk, 0)


_sc_gather = pl.kernel(
    _gather_body,
    out_type=jax.ShapeDtypeStruct((_B, _D), jnp.float32),
    mesh=plsc.VectorSubcoreMesh(core_axis_name="c", subcore_axis_name="s"),
    scratch_types=[
        pltpu.VMEM((_C,), jnp.int32),
        pltpu.VMEM((_C, _D), jnp.float32),
        pltpu.SemaphoreType.DMA,
    ],
    compiler_params=pltpu.CompilerParams(use_tc_tiling_on_sc=False),
)


@jax.jit
def kernel(tokens, embedding_weights):
    flat = tokens.reshape(-1).astype(jnp.int32)
    out = _sc_gather(flat, embedding_weights)
    return out.reshape(tokens.shape + (_D,))


# speedup vs baseline: 5.0059x; 1.0409x over previous
import jax
import jax.numpy as jnp
from jax import lax
from jax.experimental import pallas as pl
from jax.experimental.pallas import tpu as pltpu
from jax.experimental.pallas import tpu_sc as plsc

_NC = 2
_NS = 16
_NW = _NC * _NS

_BATCH = 16384
_HIST = 200
_D = 32
_RPW = _BATCH // _NW
_R = 16
_NCHUNK = _RPW // _R


def _gather_body(tokens_hbm, table_hbm, out_hbm, tok_v, rows_v, sem):
    wid = lax.axis_index("s") * _NC + lax.axis_index("c")
    base = wid * _RPW

    def chunk(i, carry):
        off = pl.multiple_of(base + i * _R, _R)
        pltpu.sync_copy(tokens_hbm.at[pl.ds(off, _R)], tok_v)
        copies = [
            pltpu.async_copy(table_hbm.at[tok_v.at[r]], rows_v.at[r], sem)
            for r in range(_R)
        ]
        for c in copies:
            c.wait()
        pltpu.sync_copy(rows_v, out_hbm.at[pl.ds(off, _R)])
        return carry

    lax.fori_loop(0, _NCHUNK, chunk, 0)


_sc_gather = pl.kernel(
    _gather_body,
    out_type=jax.ShapeDtypeStruct((_BATCH, _HIST, _D), jnp.float32),
    mesh=plsc.VectorSubcoreMesh(core_axis_name="c", subcore_axis_name="s"),
    scratch_types=[
        pltpu.VMEM((_R, _HIST), jnp.int32),
        pltpu.VMEM((_R, _HIST, _D), jnp.float32),
        pltpu.SemaphoreType.DMA,
    ],
    compiler_params=pltpu.CompilerParams(use_tc_tiling_on_sc=False),
)


@jax.jit
def kernel(tokens, embedding_weights):
    return _sc_gather(tokens.astype(jnp.int32), embedding_weights)
